# baseline (speedup 1.0000x reference)
import math

import jax
import jax.numpy as jnp
import numpy as np
from jax.experimental import pallas as pl
from jax.experimental.pallas import tpu as pltpu

_B, _N, _DIM, _H, _FACTOR = 4, 2048, 1024, 16, 5
_D = _DIM // _H
_U = min(_FACTOR * int(np.ceil(np.log(_N))), _N)
_SCALE = float(_D) ** -0.5
_QB = 256
_RB = 512

_IDX = np.asarray(jax.random.randint(jax.random.key(42), (_N, _U), 0, _N))
_CNT_T = np.zeros((_N, _N), dtype=np.float32)
np.add.at(_CNT_T, (_IDX.reshape(-1), np.repeat(np.arange(_N), _U)), 1.0)


def _qkv_kernel(x_ref, wq_ref, wk_ref, wv_ref, bq_ref, bk_ref, bv_ref,
                q_ref, k_ref, v_ref):
    x = x_ref[0]
    nt = (((1,), (1,)), ((), ()))
    for hh in range(_H):
        q_ref[0, hh] = jax.lax.dot_general(
            x, wq_ref[hh], nt,
            preferred_element_type=jnp.float32) + bq_ref[hh]
        k_ref[0, hh] = jax.lax.dot_general(
            x, wk_ref[hh], nt,
            preferred_element_type=jnp.float32) + bk_ref[hh]
        v_ref[0, hh] = jax.lax.dot_general(
            x, wv_ref[hh], nt,
            preferred_element_type=jnp.float32) + bv_ref[hh]


def _attn_kernel(q_ref, k_ref, v_ref, cntT_ref, wp_ref, bp_ref, o_ref,
                 ohs_ref, corrs_ref, vracc_ref):
    h = pl.program_id(1)
    nt = (((1,), (1,)), ((), ()))
    q = q_ref[0, 0]
    k = k_ref[0, 0]
    v = v_ref[0, 0]

    m_parts = []
    for i in range(_N // _QB):
        qb = q[i * _QB:(i + 1) * _QB]
        s_t = jax.lax.dot_general(k, qb, nt,
                                  preferred_element_type=jnp.float32)
        cf = cntT_ref[:, i * _QB:(i + 1) * _QB]
        mx = jnp.max(jnp.where(cf > 0.0, s_t, -1e30), axis=0, keepdims=True)
        ws = jnp.sum(s_t * cf, axis=0, keepdims=True)
        m_parts.append(mx - ws * (1.0 / _N))
    m = jnp.concatenate(m_parts, axis=0)

    fi = (jax.lax.broadcasted_iota(jnp.int32, (_N // _QB, _QB), 0) * _QB
          + jax.lax.broadcasted_iota(jnp.int32, (_N // _QB, _QB), 1))
    lane = jax.lax.broadcasted_iota(jnp.int32, (1, _N), 1)

    def body(j, m_cur):
        mval = jnp.max(m_cur)
        idx = jnp.min(jnp.where(m_cur == mval, fi, _N))
        ohs_ref[pl.ds(h * _U + j, 1), :] = (lane == idx).astype(jnp.float32)
        return jnp.where(fi == idx, -1e30, m_cur)

    jax.lax.fori_loop(0, _U, body, m)
    oh = ohs_ref[pl.ds(h * _U, _U), :]

    qr = jnp.dot(oh, q, preferred_element_type=jnp.float32)
    sc = jax.lax.dot_general(qr, k, nt,
                             preferred_element_type=jnp.float32) * _SCALE
    sc = sc - jnp.max(sc, axis=1, keepdims=True)
    e = jnp.exp(sc)
    attn = e / jnp.sum(e, axis=1, keepdims=True)
    upd = jnp.dot(attn, v, preferred_element_type=jnp.float32)

    vmean = jnp.mean(v, axis=0, keepdims=True)
    wp_h = wp_ref[0]
    corrs_ref[pl.ds(h * _U, _U), :] = jnp.dot(
        upd - vmean, wp_h, preferred_element_type=jnp.float32)
    vrow = jnp.dot(vmean, wp_h, preferred_element_type=jnp.float32)

    @pl.when(h == 0)
    def _first():
        vracc_ref[...] = bp_ref[...] + vrow

    @pl.when(h > 0)
    def _rest():
        vracc_ref[...] = vracc_ref[...] + vrow

    @pl.when(h == _H - 1)
    def _flush():
        o_ref[0] = vracc_ref[...] + jax.lax.dot_general(
            ohs_ref[...], corrs_ref[...], (((0,), (0,)), ((), ())),
            preferred_element_type=jnp.float32)


def kernel(x, Wq, bq, Wk, bk, Wv, bv, Wp, bp):
    Bx, Nx, C = x.shape
    wq3 = Wq.reshape(_H, _D, _DIM)
    wk3 = Wk.reshape(_H, _D, _DIM)
    wv3 = Wv.reshape(_H, _D, _DIM)
    bq3 = bq.reshape(_H, 1, _D)
    bk3 = bk.reshape(_H, 1, _D)
    bv3 = bv.reshape(_H, 1, _D)

    wspec = pl.BlockSpec((_H, _D, _DIM), lambda b, i: (0, 0, 0))
    bspec = pl.BlockSpec((_H, 1, _D), lambda b, i: (0, 0, 0))
    hshape = jax.ShapeDtypeStruct((Bx, _H, Nx, _D), jnp.float32)
    hout = pl.BlockSpec((1, _H, _RB, _D), lambda b, i: (b, 0, i, 0))
    q4, k4, v4 = pl.pallas_call(
        _qkv_kernel,
        grid=(Bx, Nx // _RB),
        in_specs=[pl.BlockSpec((1, _RB, _DIM), lambda b, i: (b, i, 0)),
                  wspec, wspec, wspec, bspec, bspec, bspec],
        out_specs=[hout, hout, hout],
        out_shape=[hshape, hshape, hshape],
    )(x, wq3, wk3, wv3, bq3, bk3, bv3)

    cntT = jnp.asarray(_CNT_T)
    wpT3 = Wp.T.reshape(_H, _D, _DIM)

    hspec = pl.BlockSpec((1, 1, _N, _D), lambda b, h: (b, h, 0, 0))
    out = pl.pallas_call(
        _attn_kernel,
        grid=(Bx, _H),
        in_specs=[
            hspec, hspec, hspec,
            pl.BlockSpec((_N, _N), lambda b, h: (0, 0)),
            pl.BlockSpec((1, _D, _DIM), lambda b, h: (h, 0, 0)),
            pl.BlockSpec((1, _DIM), lambda b, h: (0, 0)),
        ],
        out_specs=pl.BlockSpec((1, _N, _DIM), lambda b, h: (b, 0, 0)),
        out_shape=jax.ShapeDtypeStruct((Bx, Nx, C), jnp.float32),
        scratch_shapes=[pltpu.VMEM((_H * _U, _N), jnp.float32),
                        pltpu.VMEM((_H * _U, _DIM), jnp.float32),
                        pltpu.VMEM((1, _DIM), jnp.float32)],
    )(q4, k4, v4, cntT, wpT3, bp.reshape(1, _DIM))
    return out


# speedup vs baseline: 1.1273x; 1.1273x over previous
import math

import jax
import jax.numpy as jnp
import numpy as np
from jax.experimental import pallas as pl
from jax.experimental.pallas import tpu as pltpu

_B, _N, _DIM, _H, _FACTOR = 4, 2048, 1024, 16, 5
_D = _DIM // _H
_U = min(_FACTOR * int(np.ceil(np.log(_N))), _N)
_SCALE = float(_D) ** -0.5
_QB = 256
_RB = 512

_IDX = np.asarray(jax.random.randint(jax.random.key(42), (_N, _U), 0, _N))
_CNT_T = np.zeros((_N, _N), dtype=np.float32)
np.add.at(_CNT_T, (_IDX.reshape(-1), np.repeat(np.arange(_N), _U)), 1.0)


def _qkv_kernel(x_ref, wq_ref, wk_ref, wv_ref, bq_ref, bk_ref, bv_ref,
                q_ref, k_ref, v_ref):
    x = x_ref[0]
    tt = (((1,), (1,)), ((), ()))
    q_ref[0] = jax.lax.dot_general(
        wq_ref[...], x, tt, preferred_element_type=jnp.float32) + bq_ref[...]
    k_ref[0] = jax.lax.dot_general(
        wk_ref[...], x, tt, preferred_element_type=jnp.float32) + bk_ref[...]
    v_ref[0] = jax.lax.dot_general(
        wv_ref[...], x, tt, preferred_element_type=jnp.float32) + bv_ref[...]


def _attn_kernel(qt_ref, kt_ref, vt_ref, cntT_ref, wp_ref, bp_ref, o_ref,
                 ohs_ref, corrs_ref, vracc_ref):
    h = pl.program_id(1)
    qt = qt_ref[0, 0]
    kt = kt_ref[0, 0]
    vt = vt_ref[0, 0]

    m_parts = []
    for i in range(_N // _QB):
        qtb = qt[:, i * _QB:(i + 1) * _QB]
        s_t = jax.lax.dot_general(kt, qtb, (((0,), (0,)), ((), ())),
                                  preferred_element_type=jnp.float32)
        cf = cntT_ref[:, i * _QB:(i + 1) * _QB]
        mx = jnp.max(jnp.where(cf > 0.0, s_t, -1e30), axis=0, keepdims=True)
        ws = jnp.sum(s_t * cf, axis=0, keepdims=True)
        m_parts.append(mx - ws * (1.0 / _N))
    m = jnp.concatenate(m_parts, axis=0)

    fi = (jax.lax.broadcasted_iota(jnp.int32, (_N // _QB, _QB), 0) * _QB
          + jax.lax.broadcasted_iota(jnp.int32, (_N // _QB, _QB), 1))
    lane = jax.lax.broadcasted_iota(jnp.int32, (1, _N), 1)

    def body(j, m_cur):
        mval = jnp.max(m_cur)
        idx = jnp.min(jnp.where(m_cur == mval, fi, _N))
        ohs_ref[pl.ds(h * _U + j, 1), :] = (lane == idx).astype(jnp.float32)
        return jnp.where(fi == idx, -1e30, m_cur)

    jax.lax.fori_loop(0, _U, body, m)
    oh = ohs_ref[pl.ds(h * _U, _U), :]

    qr = jax.lax.dot_general(oh, qt, (((1,), (1,)), ((), ())),
                             preferred_element_type=jnp.float32)
    sc = jax.lax.dot_general(qr, kt, (((1,), (0,)), ((), ())),
                             preferred_element_type=jnp.float32) * _SCALE
    sc = sc - jnp.max(sc, axis=1, keepdims=True)
    e = jnp.exp(sc)
    attn = e / jnp.sum(e, axis=1, keepdims=True)
    upd = jax.lax.dot_general(attn, vt, (((1,), (1,)), ((), ())),
                              preferred_element_type=jnp.float32)

    vmean = jax.lax.dot_general(jnp.full((1, _N), 1.0 / _N, jnp.float32),
                                vt, (((1,), (1,)), ((), ())),
                                preferred_element_type=jnp.float32)
    wp_h = wp_ref[0]
    corrs_ref[pl.ds(h * _U, _U), :] = jnp.dot(
        upd - vmean, wp_h, preferred_element_type=jnp.float32)
    vrow = jnp.dot(vmean, wp_h, preferred_element_type=jnp.float32)

    @pl.when(h == 0)
    def _first():
        vracc_ref[...] = bp_ref[...] + vrow

    @pl.when(h > 0)
    def _rest():
        vracc_ref[...] = vracc_ref[...] + vrow

    @pl.when(h == _H - 1)
    def _flush():
        o_ref[0] = vracc_ref[...] + jax.lax.dot_general(
            ohs_ref[...], corrs_ref[...], (((0,), (0,)), ((), ())),
            preferred_element_type=jnp.float32)


def kernel(x, Wq, bq, Wk, bk, Wv, bv, Wp, bp):
    Bx, Nx, C = x.shape
    wspec = pl.BlockSpec((_DIM, _DIM), lambda b, i: (0, 0))
    bspec = pl.BlockSpec((_DIM, 1), lambda b, i: (0, 0))
    tshape = jax.ShapeDtypeStruct((Bx, _DIM, Nx), jnp.float32)
    tout = pl.BlockSpec((1, _DIM, _RB), lambda b, i: (b, 0, i))
    qt, kt, vt = pl.pallas_call(
        _qkv_kernel,
        grid=(Bx, Nx // _RB),
        in_specs=[pl.BlockSpec((1, _RB, _DIM), lambda b, i: (b, i, 0)),
                  wspec, wspec, wspec, bspec, bspec, bspec],
        out_specs=[tout, tout, tout],
        out_shape=[tshape, tshape, tshape],
    )(x, Wq, Wk, Wv, bq.reshape(_DIM, 1), bk.reshape(_DIM, 1),
      bv.reshape(_DIM, 1))

    qt4 = qt.reshape(Bx, _H, _D, Nx)
    kt4 = kt.reshape(Bx, _H, _D, Nx)
    vt4 = vt.reshape(Bx, _H, _D, Nx)

    cntT = jnp.asarray(_CNT_T)
    wpT3 = Wp.T.reshape(_H, _D, _DIM)

    hspec = pl.BlockSpec((1, 1, _D, _N), lambda b, h: (b, h, 0, 0))
    out = pl.pallas_call(
        _attn_kernel,
        grid=(Bx, _H),
        in_specs=[
            hspec, hspec, hspec,
            pl.BlockSpec((_N, _N), lambda b, h: (0, 0)),
            pl.BlockSpec((1, _D, _DIM), lambda b, h: (h, 0, 0)),
            pl.BlockSpec((1, _DIM), lambda b, h: (0, 0)),
        ],
        out_specs=pl.BlockSpec((1, _N, _DIM), lambda b, h: (b, 0, 0)),
        out_shape=jax.ShapeDtypeStruct((Bx, Nx, C), jnp.float32),
        scratch_shapes=[pltpu.VMEM((_H * _U, _N), jnp.float32),
                        pltpu.VMEM((_H * _U, _DIM), jnp.float32),
                        pltpu.VMEM((1, _DIM), jnp.float32)],
    )(qt4, kt4, vt4, cntT, wpT3, bp.reshape(1, _DIM))
    return out
